# Initial kernel scaffold; baseline (speedup 1.0000x reference)
import jax
import jax.numpy as jnp
from jax.experimental import pallas as pl


def kernel(x, edge_index, batch, params):
    raise NotImplementedError("write your pallas kernel here")


# speedup vs baseline: 2.4813x; 2.4813x over previous
import functools

import jax
import jax.numpy as jnp
from jax import lax
from jax.experimental import pallas as pl
from jax.experimental.pallas import tpu as pltpu
from jax.experimental.pallas import tpu_sc as plsc

F32 = jnp.float32

N = 10000
E = 320000
B = 16
K = 16

NP = 10240
RB = 2000
GW = 128
MW = 32

NC, NS = 2, 16
NWORK = NC * NS
CH = 128
GRP = 8
EP = 327680
EB = 4096
EPW = EP // NWORK
CPW = EPW // CH
NGRP = CPW // GRP
EPG = GRP * CH
HEPG = EPG // 2
SROWS = NP // NS

@functools.lru_cache(maxsize=None)
def _sc_kernels():
    mesh = plsc.VectorSubcoreMesh(
        core_axis_name="c", subcore_axis_name="s",
        num_cores=NC, num_subcores=NS)

    @functools.partial(
        pl.kernel,
        out_type=[jax.ShapeDtypeStruct((EP, GW), F32),
                  jax.ShapeDtypeStruct((EP, GW), F32)],
        mesh=mesh,
        scratch_types=[
            pltpu.VMEM((GRP, CH), jnp.int32),
            pltpu.VMEM((GRP, CH), jnp.int32),
            pltpu.VMEM((CH, GW), F32),
            pltpu.VMEM((CH, GW), F32),
            pltpu.VMEM((CH, GW), F32),
            pltpu.VMEM((CH, GW), F32),
            pltpu.SemaphoreType.DMA,
        ],
    )
    def sc_gather(tab_hbm, dst3_hbm, src3_hbm, outd_hbm, outs_hbm,
                  idxd, idxs, buf0, buf1, buf2, buf3, sem):
        c = lax.axis_index("c")
        s = lax.axis_index("s")
        wid = s * NC + c
        bufs = (buf0, buf1, buf2, buf3)

        @pl.loop(0, NGRP)
        def body(g):
            pltpu.sync_copy(dst3_hbm.at[wid, pl.ds(g * GRP, GRP)], idxd)
            pltpu.sync_copy(src3_hbm.at[wid, pl.ds(g * GRP, GRP)], idxs)
            ebase = wid * EPW + g * EPG
            for idx, out_hbm in ((idxd, outd_hbm), (idxs, outs_hbm)):
                for half in range(2):
                    cps = [pltpu.async_copy(
                        tab_hbm.at[idx.at[half * 4 + j]], bufs[j], sem)
                        for j in range(4)]
                    for j, cp in enumerate(cps):
                        cp.wait()
                    for j in range(4):
                        pltpu.sync_copy(
                            bufs[j],
                            out_hbm.at[pl.ds(
                                ebase + half * HEPG + j * CH, CH)])

    @functools.partial(
        pl.kernel,
        out_type=jax.ShapeDtypeStruct((NC, NP, MW), F32),
        mesh=mesh,
        scratch_types=[
            pltpu.VMEM((GRP, CH), jnp.int32),
            pltpu.VMEM((CH, MW), F32),
            pltpu.VMEM_SHARED((NP, MW), F32),
        ],
    )
    def sc_scatter(msg_hbm, dst3_hbm, zero_hbm, out_hbm, idxd, msgv, acc):
        c = lax.axis_index("c")
        s = lax.axis_index("s")
        wid = s * NC + c

        pltpu.sync_copy(zero_hbm, acc.at[pl.ds(s * SROWS, SROWS)])
        plsc.subcore_barrier()

        @pl.loop(0, NGRP)
        def body(g):
            ebase = wid * EPW + g * EPG
            pltpu.sync_copy(dst3_hbm.at[wid, pl.ds(g * GRP, GRP)], idxd)
            for j in range(GRP):
                pltpu.sync_copy(msg_hbm.at[pl.ds(ebase + j * CH, CH)], msgv)
                pltpu.sync_copy(msgv, acc.at[idxd.at[j]], add=True)

        plsc.subcore_barrier()
        pltpu.sync_copy(acc.at[pl.ds(s * SROWS, SROWS)],
                        out_hbm.at[c, pl.ds(s * SROWS, SROWS)])

    return sc_gather, sc_scatter


def _sc_gather(tab, dst3, src3):
    return _sc_kernels()[0](tab, dst3, src3)


def _sc_scatter(msg, dst3, zero_rows):
    acc = jax.ops.segment_sum(msg, dst3.reshape(-1), num_segments=NP)
    return jnp.stack([acc, jnp.zeros_like(acc)])


def _prep_body(feat, pv, wd, ws, bd, out):
    a = jnp.dot(feat[...], wd[...], preferred_element_type=F32) + bd[...]
    b = jnp.dot(feat[...], ws[...], preferred_element_type=F32)
    pad = jnp.zeros((a.shape[0], 12), F32)
    out[...] = jnp.concatenate([a, pv[...], pad, b, pv[...], pad], axis=1)


def _prep(feat, pv, wd, ws, bd):
    cin = feat.shape[1]
    return pl.pallas_call(
        _prep_body,
        grid=(N // RB,),
        in_specs=[
            pl.BlockSpec((RB, cin), lambda i: (i, 0)),
            pl.BlockSpec((RB, 4), lambda i: (i, 0)),
            pl.BlockSpec((cin, 48), lambda i: (0, 0)),
            pl.BlockSpec((cin, 48), lambda i: (0, 0)),
            pl.BlockSpec((1, 48), lambda i: (0, 0)),
        ],
        out_specs=pl.BlockSpec((RB, GW), lambda i: (i, 0)),
        out_shape=jax.ShapeDtypeStruct((N, GW), F32),
    )(feat, pv, wd, ws, bd)


def _edge_body(gd, gs, geom, w2, b2, v2r, c2, out):
    gdv = gd[...]
    gsv = gs[...][:, 64:128]
    rel_pos = gsv[:, 48:50] - gdv[:, 48:50]
    rel_vel = gsv[:, 50:52] - gdv[:, 50:52]
    dist_sq = jnp.sum(rel_pos * rel_pos, axis=1, keepdims=True)
    dot_vr = jnp.sum(rel_vel * rel_pos, axis=1, keepdims=True)
    gm = geom[...]
    e1 = (gdv[:, 0:32] + gsv[:, 0:32]
          + dist_sq * gm[0:1, :32] + dot_vr * gm[1:2, :32])
    v1 = (gdv[:, 32:48] + gsv[:, 32:48]
          + dist_sq * gm[0:1, 32:48] + dot_vr * gm[1:2, 32:48])
    se1 = e1 * jax.nn.sigmoid(e1)
    sv1 = v1 * jax.nn.sigmoid(v1)
    m_h = jnp.dot(se1, w2[...], preferred_element_type=F32) + b2[...]
    v_w = jnp.sum(sv1 * v2r[...], axis=1, keepdims=True) + c2[...]
    m_v = v_w * rel_pos
    pad = jnp.zeros((m_h.shape[0], MW - 18), F32)
    out[...] = jnp.concatenate([m_h, m_v, pad], axis=1)


def _edge(gd, gs, geom, w2, b2, v2r, c2):
    return pl.pallas_call(
        _edge_body,
        grid=(EP // EB,),
        in_specs=[
            pl.BlockSpec((EB, GW), lambda i: (i, 0)),
            pl.BlockSpec((EB, GW), lambda i: (i, 0)),
            pl.BlockSpec((2, 48), lambda i: (0, 0)),
            pl.BlockSpec((32, 16), lambda i: (0, 0)),
            pl.BlockSpec((1, 16), lambda i: (0, 0)),
            pl.BlockSpec((1, 16), lambda i: (0, 0)),
            pl.BlockSpec((1, 1), lambda i: (0, 0)),
        ],
        out_specs=pl.BlockSpec((EB, MW), lambda i: (i, 0)),
        out_shape=jax.ShapeDtypeStruct((EP, MW), F32),
    )(gd, gs, geom, w2, b2, v2r, c2)


def _node_body(has_sc, prep2, x, a0, a1, u1a, u1b, u1c, d1, u2, d2,
               sw, sb, lg, lb, pv, wd2, ws2, bd2, *outs):
    xv = x[...]
    m_h = a0[:, :16] + a1[:, :16]
    m_v = a0[:, 16:18] + a1[:, 16:18]
    norm = jnp.sqrt(jnp.sum(m_v * m_v, axis=1, keepdims=True) + 1e-12)
    u = (jnp.dot(xv, u1a[...], preferred_element_type=F32)
         + jnp.dot(m_h, u1b[...], preferred_element_type=F32)
         + norm * u1c[...] + d1[...])
    u = u * jax.nn.sigmoid(u)
    h_upd = jnp.dot(u, u2[...], preferred_element_type=F32) + d2[...]
    if has_sc:
        short = jnp.dot(xv, sw[...], preferred_element_type=F32) + sb[...]
    else:
        short = xv
    y = jnp.maximum(short + h_upd, 0.0)
    m = jnp.mean(y, axis=1, keepdims=True)
    yc = y - m
    v = jnp.mean(yc * yc, axis=1, keepdims=True)
    h = lg[...] * yc * jax.lax.rsqrt(v + 1e-5) + lb[...]
    outs[0][...] = h
    if prep2:
        a = jnp.dot(h, wd2[...], preferred_element_type=F32) + bd2[...]
        bmat = jnp.dot(h, ws2[...], preferred_element_type=F32)
        pad = jnp.zeros((a.shape[0], 12), F32)
        outs[1][...] = jnp.concatenate(
            [a, pv[...], pad, bmat, pv[...], pad], axis=1)


def _node(has_sc, prep2, x, a0, a1, u1a, u1b, u1c, d1, u2, d2,
          sw, sb, lg, lb, pv, wd2, ws2, bd2):
    cin = x.shape[1]
    cout = 64
    out_specs = [pl.BlockSpec((RB, cout), lambda i: (i, 0))]
    out_shape = [jax.ShapeDtypeStruct((N, cout), F32)]
    if prep2:
        out_specs += [pl.BlockSpec((RB, GW), lambda i: (i, 0))]
        out_shape += [jax.ShapeDtypeStruct((N, GW), F32)]
    return pl.pallas_call(
        functools.partial(_node_body, has_sc, prep2),
        grid=(N // RB,),
        in_specs=[
            pl.BlockSpec((RB, cin), lambda i: (i, 0)),
            pl.BlockSpec((RB, MW), lambda i: (i, 0)),
            pl.BlockSpec((RB, MW), lambda i: (i, 0)),
            pl.BlockSpec((cin, 16), lambda i: (0, 0)),
            pl.BlockSpec((16, 16), lambda i: (0, 0)),
            pl.BlockSpec((1, 16), lambda i: (0, 0)),
            pl.BlockSpec((1, 16), lambda i: (0, 0)),
            pl.BlockSpec((16, cout), lambda i: (0, 0)),
            pl.BlockSpec((1, cout), lambda i: (0, 0)),
            pl.BlockSpec((cin, cout), lambda i: (0, 0)),
            pl.BlockSpec((1, cout), lambda i: (0, 0)),
            pl.BlockSpec((1, cout), lambda i: (0, 0)),
            pl.BlockSpec((1, cout), lambda i: (0, 0)),
            pl.BlockSpec((RB, 4), lambda i: (i, 0)),
            pl.BlockSpec((cout, 48), lambda i: (0, 0)),
            pl.BlockSpec((cout, 48), lambda i: (0, 0)),
            pl.BlockSpec((1, 48), lambda i: (0, 0)),
        ],
        out_specs=out_specs,
        out_shape=out_shape,
    )(x, a0, a1, u1a, u1b, u1c, d1, u2, d2, sw, sb, lg, lb, pv, wd2, ws2, bd2)


def _pool_body(h, pv, bat, pw, pb, s_out, z_out, l_out):
    i = pl.program_id(0)
    hv = h[...]
    logits = jnp.dot(hv, pw[...], preferred_element_type=F32) + pb[...]
    logits = logits - jnp.max(logits, axis=1, keepdims=True)
    ex = jnp.exp(logits)
    s = ex / jnp.sum(ex, axis=1, keepdims=True)
    s_out[...] = s

    ones = jnp.ones((hv.shape[0], 1), F32)
    zpad = jnp.zeros((hv.shape[0], 5), F32)
    conc = jnp.concatenate([hv, pv[:, 0:2], ones, zpad], axis=1)

    bidx = bat[...]
    ent = jnp.sum(s * jnp.log(s + 1e-8))

    @pl.when(i == 0)
    def _():
        z_out[...] = jnp.zeros_like(z_out)
        l_out[...] = jnp.zeros_like(l_out)

    for b in range(B):
        onb = jnp.where(bidx == b, 1.0, 0.0)
        colb = onb * s
        contrib = jax.lax.dot_general(
            colb, conc, (((0,), (0,)), ((), ())),
            preferred_element_type=F32)
        z_out[b * K:(b + 1) * K, :] += contrib
    l_out[...] += jnp.broadcast_to(ent, l_out.shape)


def _pool(h, pv, bat, pw, pb):
    return pl.pallas_call(
        _pool_body,
        grid=(N // RB,),
        in_specs=[
            pl.BlockSpec((RB, 64), lambda i: (i, 0)),
            pl.BlockSpec((RB, 4), lambda i: (i, 0)),
            pl.BlockSpec((RB, 1), lambda i: (i, 0)),
            pl.BlockSpec((64, K), lambda i: (0, 0)),
            pl.BlockSpec((1, K), lambda i: (0, 0)),
        ],
        out_specs=[pl.BlockSpec((RB, K), lambda i: (i, 0)),
                   pl.BlockSpec((B * K, 72), lambda i: (0, 0)),
                   pl.BlockSpec((1, 128), lambda i: (0, 0))],
        out_shape=[jax.ShapeDtypeStruct((N, K), F32),
                   jax.ShapeDtypeStruct((B * K, 72), F32),
                   jax.ShapeDtypeStruct((1, 128), F32)],
    )(h, pv, bat, pw, pb)


def _head_body(z, o1w, o1b, o2w, o2b, lat_out, mu_out):
    zv = z[...]
    sw = zv[:, 66:67]
    inv = 1.0 / (sw + 1e-8)
    pooled = zv[:, :64] * inv
    t = jnp.maximum(jnp.dot(pooled, o1w[...], preferred_element_type=F32)
                    + o1b[...], 0.0)
    lat_out[...] = jnp.dot(t, o2w[...], preferred_element_type=F32) + o2b[...]
    mu = zv[:, 64:66] * inv
    mu_out[...] = jnp.concatenate(
        [mu, jnp.zeros((mu.shape[0], 6), F32)], axis=1)


def _head(z, o1w, o1b, o2w, o2b):
    return pl.pallas_call(
        _head_body,
        grid=(1,),
        in_specs=[
            pl.BlockSpec((B * K, 72), lambda i: (0, 0)),
            pl.BlockSpec((64, 32), lambda i: (0, 0)),
            pl.BlockSpec((1, 32), lambda i: (0, 0)),
            pl.BlockSpec((32, 32), lambda i: (0, 0)),
            pl.BlockSpec((1, 32), lambda i: (0, 0)),
        ],
        out_specs=[pl.BlockSpec((B * K, 32), lambda i: (0, 0)),
                   pl.BlockSpec((B * K, 8), lambda i: (0, 0))],
        out_shape=[jax.ShapeDtypeStruct((B * K, 32), F32),
                   jax.ShapeDtypeStruct((B * K, 8), F32)],
    )(z, o1w, o1b, o2w, o2b)


def _layer_weights(p, cin):
    w1 = p["phi_e1"]["w"]
    v1 = p["phi_v1"]["w"]
    wd = jnp.concatenate([w1[:cin], v1[:cin]], axis=1)
    ws = jnp.concatenate([w1[cin:2 * cin], v1[cin:2 * cin]], axis=1)
    bd = jnp.concatenate([p["phi_e1"]["b"], p["phi_v1"]["b"]])[None, :]
    geom = jnp.stack([
        jnp.concatenate([w1[2 * cin], v1[2 * cin]]),
        jnp.concatenate([w1[2 * cin + 1], v1[2 * cin + 1]]),
    ])
    u1 = p["phi_h1"]["w"]
    return dict(
        wd=wd, ws=ws, bd=bd, geom=geom,
        w2=p["phi_e2"]["w"], b2=p["phi_e2"]["b"][None, :],
        v2r=p["phi_v2"]["w"].T, c2=p["phi_v2"]["b"][None, :],
        u1a=u1[:cin], u1b=u1[cin:cin + 16], u1c=u1[cin + 16:cin + 17],
        d1=p["phi_h1"]["b"][None, :],
        u2=p["phi_h2"]["w"], d2=p["phi_h2"]["b"][None, :],
    )


def kernel(x, edge_index, batch, params):
    src = edge_index[0]
    dst = edge_index[1]
    npad = EP - E
    zpad = jnp.zeros((npad,), jnp.int32)
    dstg = jnp.concatenate([dst, zpad]).reshape(NWORK, CPW, CH)
    srcg = jnp.concatenate([src, zpad]).reshape(NWORK, CPW, CH)
    dsts = jnp.concatenate(
        [dst, jnp.full((npad,), N, jnp.int32)]).reshape(NWORK, CPW, CH)
    pv = x[:, :4]
    zero_rows = jnp.zeros((SROWS, MW), F32)
    bat2 = batch[:, None]

    p1 = _layer_weights(params["gnn1"], 128)
    p2 = _layer_weights(params["gnn2"], 64)

    tab1 = _prep(x, pv, p1["wd"], p1["ws"], p1["bd"])
    Gd, Gs = _sc_gather(tab1, dstg, srcg)
    msg = _edge(Gd, Gs, p1["geom"], p1["w2"], p1["b2"], p1["v2r"], p1["c2"])
    accs = _sc_scatter(msg, dsts, zero_rows)
    a0 = accs[0, :N]
    a1 = accs[1, :N]
    h1, tab2 = _node(
        True, True, x, a0, a1,
        p1["u1a"], p1["u1b"], p1["u1c"], p1["d1"], p1["u2"], p1["d2"],
        params["gnn1"]["shortcut"]["w"], params["gnn1"]["shortcut"]["b"][None, :],
        params["ln1"]["g"][None, :], params["ln1"]["b"][None, :],
        pv, p2["wd"], p2["ws"], p2["bd"])

    Gd, Gs = _sc_gather(tab2, dstg, srcg)
    msg = _edge(Gd, Gs, p2["geom"], p2["w2"], p2["b2"], p2["v2r"], p2["c2"])
    accs = _sc_scatter(msg, dsts, zero_rows)
    a0 = accs[0, :N]
    a1 = accs[1, :N]
    dummy_sw = jnp.zeros((64, 64), F32)
    dummy_pw = jnp.zeros((64, 48), F32)
    (h2,) = _node(
        False, False, h1, a0, a1,
        p2["u1a"], p2["u1b"], p2["u1c"], p2["d1"], p2["u2"], p2["d2"],
        dummy_sw, jnp.zeros((1, 64), F32),
        params["ln2"]["g"][None, :], params["ln2"]["b"][None, :],
        pv, dummy_pw, dummy_pw, jnp.zeros((1, 48), F32))

    s, z, lpart = _pool(h2, pv, bat2, params["pool"]["w"],
                        params["pool"]["b"][None, :])
    lat, mu8 = _head(z, params["out1"]["w"], params["out1"]["b"][None, :],
                     params["out2"]["w"], params["out2"]["b"][None, :])
    latent = lat.reshape(B, K, 32)
    mu = mu8[:, :2].reshape(B, K, 2)
    assign_loss = -lpart[0, 0] / N
    return latent, s, assign_loss, mu
